# Initial kernel scaffold; baseline (speedup 1.0000x reference)
import jax
import jax.numpy as jnp
from jax.experimental import pallas as pl


def kernel(x, edge_index, W_dist, b_dist, W_self, W_neigh, b_sage, W_out, b_out):
    raise NotImplementedError("write your pallas kernel here")


# speedup vs baseline: 1.0576x; 1.0576x over previous
import functools
import jax
import jax.numpy as jnp
from jax.experimental import pallas as pl
from jax.experimental.pallas import tpu as pltpu


def _final_body(x_ref, nm_ref, ws_ref, wn_ref, bs_ref, wo_ref, bo_ref, out_ref):
    h = jnp.dot(x_ref[...], ws_ref[...], preferred_element_type=jnp.float32)
    h += jnp.dot(nm_ref[...], wn_ref[...], preferred_element_type=jnp.float32)
    h += bs_ref[...]
    h = jnp.maximum(h, 0.0)
    out_ref[...] = jnp.dot(h, wo_ref[...], preferred_element_type=jnp.float32) + bo_ref[...]


def _final_matmuls(x, neigh_mean, W_self, W_neigh, b_sage, W_out_pad, b_out_pad):
    n, d = x.shape
    blk = 1000
    grid = (n // blk,)
    return pl.pallas_call(
        _final_body,
        grid=grid,
        in_specs=[
            pl.BlockSpec((blk, d), lambda i: (i, 0)),
            pl.BlockSpec((blk, d), lambda i: (i, 0)),
            pl.BlockSpec(W_self.shape, lambda i: (0, 0)),
            pl.BlockSpec(W_neigh.shape, lambda i: (0, 0)),
            pl.BlockSpec(b_sage.shape, lambda i: (0, 0)),
            pl.BlockSpec(W_out_pad.shape, lambda i: (0, 0)),
            pl.BlockSpec(b_out_pad.shape, lambda i: (0, 0)),
        ],
        out_specs=pl.BlockSpec((blk, 128), lambda i: (i, 0)),
        out_shape=jax.ShapeDtypeStruct((n, 128), jnp.float32),
    )(x, neigh_mean, W_self, W_neigh, b_sage, W_out_pad, b_out_pad)


def kernel(x, edge_index, W_dist, b_dist, W_self, W_neigh, b_sage, W_out, b_out):
    src = edge_index[0]
    dst = edge_index[1]
    n = x.shape[0]
    e = src.shape[0]

    s = x @ W_dist + b_dist
    sig = jax.nn.sigmoid(s[:, 0])

    diff = jnp.abs(sig[dst] - sig[src])
    order = jnp.lexsort((diff, dst))
    dst_s = dst[order]
    src_s = src[order]
    deg = jnp.bincount(dst, length=n)
    start = jnp.concatenate([jnp.zeros((1,), dtype=deg.dtype), jnp.cumsum(deg)[:-1]])
    rank = jnp.arange(e) - start[dst_s]
    k = jnp.maximum(deg // 2, 1)
    keep = (rank < k[dst_s]).astype(x.dtype)

    msg = x[src_s] * keep[:, None]
    neigh_sum = jnp.zeros((n, x.shape[1]), dtype=x.dtype).at[dst_s].add(msg)
    cnt = jnp.zeros((n,), dtype=x.dtype).at[dst_s].add(keep)
    neigh_mean = neigh_sum / jnp.maximum(cnt, 1.0)[:, None]

    b_sage2 = jnp.broadcast_to(b_sage[None, :], (1, 128))
    W_out_pad = jnp.zeros((128, 128), jnp.float32).at[:, :2].set(W_out)
    b_out_pad = jnp.zeros((1, 128), jnp.float32).at[0, :2].set(b_out)
    out = _final_matmuls(x, neigh_mean, W_self, W_neigh, b_sage2, W_out_pad, b_out_pad)
    logits = out[:, :2]
    return (logits, s)


# speedup vs baseline: 4.9770x; 4.7060x over previous
import jax
import jax.numpy as jnp
from jax import lax
from jax.experimental import pallas as pl
from jax.experimental.pallas import tpu as pltpu
from jax.experimental.pallas import tpu_sc as plsc

N = 10000
D = 128
NPAD = 10240
NR = NPAD // 128
NW = 16
NPW = NPAD // NW
EPAD = 327680
ER = EPAD // 128
EB_W = ER // NW
HPADS = 2048
HSIZE = 16 * NPAD + HPADS
NQ = NPAD // 4


def _i16(v):
    return jnp.full((16,), v, jnp.int32)


def _iota16():
    return lax.iota(jnp.int32, 16)


def _gat(ref, idx):
    return plsc.load_gather(ref, [lax.shift_right_logical(idx, _i16(7)),
                                  idx & 127])


def _gat_pt(ref, v):
    wq = v // 640
    flat = wq * 1024 + (v - wq * 640)
    return _gat(ref, flat)


def _rowg(ref, row, q):
    return plsc.load_gather(ref, [_i16(row), q * 16 + _iota16()])


def _rows(ref, row, q, val, mask=None):
    plsc.store_scatter(ref, [_i16(row), q * 16 + _iota16()], val, mask=mask)


def _select_body(sig_hbm, src_hbm, dst_hbm,
                 key_hbm, t_hbm, i_hbm, deg_hbm,
                 sigT, dstv, keyv, srcv, pLocal, hidxv, onesv, histv,
                 zbufi, rv, Pv, Tv,
                 hists, degs, Ps, Ts, sigS):
    w = lax.axis_index("s")
    nbase = w * NPW
    z16 = jnp.zeros((16,), jnp.int32)

    def _fill_zb(i, _):
        zbufi[pl.ds(i * 16, 16)] = jnp.zeros((16,), jnp.int32)
        return 0
    lax.fori_loop(0, NPW // 16, _fill_zb, 0)

    for q in range(8):
        onesv[pl.ds(q * 16, 16)] = jnp.ones((16,), jnp.int32)

    for r in range(8):
        for q in range(8):
            Pv[r, pl.ds(q * 16, 16)] = jnp.zeros((16,), jnp.int32)
            Tv[r, pl.ds(q * 16, 16)] = jnp.zeros((16,), jnp.int32)

    pltpu.sync_copy(zbufi, degs.at[pl.ds(nbase, NPW)])

    def _zero_hist(z, _):
        pltpu.sync_copy(zbufi, hists.at[pl.ds(z * NPAD + nbase, NPW)])
        return 0
    lax.fori_loop(0, 16, _zero_hist, 0)
    pltpu.sync_copy(zbufi.at[pl.ds(0, 128)],
                    hists.at[pl.ds(16 * NPAD + w * 128, 128)])

    pltpu.sync_copy(Pv, Ps.at[pl.ds(w * 8, 8), :])
    pltpu.sync_copy(Tv, Ts.at[pl.ds(w * 8, 8), :])

    @pl.when(w == 0)
    def _stage_sig():
        pltpu.sync_copy(sig_hbm, sigS)
    plsc.subcore_barrier()

    pltpu.sync_copy(sigS, sigT.at[pl.ds(0, NR), :])

    def _load_dst(c, _):
        pltpu.sync_copy(dst_hbm.at[pl.ds(w * EB_W + c * 32, 32), :],
                        dstv.at[pl.ds(c * 32, 32), :])
        return 0
    lax.fori_loop(0, 5, _load_dst, 0)

    def _chunk_keys(c, _):
        pltpu.sync_copy(src_hbm.at[pl.ds(w * EB_W + c * 32, 32), :], srcv)

        def _keys(eb, _):
            row = c * 32 + eb

            def _q(q, _):
                s16 = _rowg(srcv, eb, q)
                d16 = _rowg(dstv, row, q)
                ss = plsc.bitcast(_gat(sigT, s16), jnp.float32)
                sd = plsc.bitcast(_gat(sigT, d16), jnp.float32)
                df = jnp.abs(sd - ss)
                _rows(keyv, row, q, plsc.bitcast(df, jnp.int32))
                return 0
            lax.fori_loop(0, 8, _q, 0)
            return 0
        lax.fori_loop(0, 32, _keys, 0)
        pltpu.sync_copy(keyv.at[pl.ds(c * 32, 32), :],
                        key_hbm.at[pl.ds(w * EB_W + c * 32, 32), :])
        return 0
    lax.fori_loop(0, 5, _chunk_keys, 0)

    def _deg(eb, _):
        def _q(q, _):
            hidxv[0, pl.ds(q * 16, 16)] = _rowg(dstv, eb, q)
            return 0
        lax.fori_loop(0, 8, _q, 0)
        pltpu.sync_copy(onesv, degs.at[hidxv.at[0]], add=True)
        return 0
    lax.fori_loop(0, EB_W, _deg, 0)
    plsc.subcore_barrier()

    pltpu.sync_copy(degs.at[pl.ds(nbase, NPW)], rv)

    def _initr(g, _):
        d16 = rv[pl.ds(g * 16, 16)]
        rv[pl.ds(g * 16, 16)] = jnp.maximum(
            lax.shift_right_logical(d16, _i16(1)), 1)
        return 0
    lax.fori_loop(0, NPW // 16, _initr, 0)

    def _node_epilogue():
        def _load_hist(b, _):
            pltpu.sync_copy(hists.at[pl.ds(b * NPAD + nbase, NPW)],
                            histv.at[pl.ds(b * NPW, NPW)])
            return 0
        lax.fori_loop(0, 16, _load_hist, 0)
        for r in range(5):
            def _c8(c8, _):
                off = r * 128 + c8 * 16
                r16 = rv[pl.ds(off, 16)]
                cum = z16
                t = z16
                cb = z16
                for b in range(16):
                    c16 = histv[pl.ds(b * NPW + off, 16)]
                    cum = cum + c16
                    cond = cum < r16
                    t = t + cond.astype(jnp.int32)
                    cb = jnp.where(cond, cum, cb)
                Pv[r, pl.ds(c8 * 16, 16)] = Pv[r, pl.ds(c8 * 16, 16)] * 16 + t
                rv[pl.ds(off, 16)] = r16 - cb
                return 0
            lax.fori_loop(0, 8, _c8, 0)

        def _rezero_hist(b, _):
            pltpu.sync_copy(zbufi, hists.at[pl.ds(b * NPAD + nbase, NPW)])
            return 0
        lax.fori_loop(0, 16, _rezero_hist, 0)
        pltpu.sync_copy(zbufi.at[pl.ds(0, 128)],
                        hists.at[pl.ds(16 * NPAD + w * 128, 128)])
        pltpu.sync_copy(Pv, Ps.at[pl.ds(w * 8, 8), :])

    def _edge_scan(eb, key_mode, sv):
        def _q(q, _):
            k16 = _rowg(keyv, eb, q)
            d16 = _rowg(dstv, eb, q)
            p16 = _gat_pt(pLocal, d16)
            if key_mode:
                kk = k16
                act = lax.shift_right_logical(k16, _i16(sv + 4)) == p16
            else:
                t16 = _gat_pt(sigT, d16)
                kk = (w * EB_W + eb) * 128 + q * 16 + _iota16()
                act = (k16 == t16) & (
                    lax.shift_right_logical(kk, _i16(sv + 4)) == p16)
            dig = lax.shift_right_logical(kk, _i16(sv)) & 15
            pad = 16 * NPAD + ((eb * 128 + q * 16 + _iota16()) & (HPADS - 1))
            hidxv[0, pl.ds(q * 16, 16)] = jnp.where(
                act, dig * NPAD + d16, pad)
            return 0
        lax.fori_loop(0, 8, _q, 0)
        pltpu.sync_copy(onesv, hists.at[hidxv.at[0]], add=True)

    def _key_pass(p, _):
        sv = 28 - 4 * p
        pltpu.sync_copy(Ps, pLocal)

        def _eb(eb, _):
            _edge_scan(eb, True, sv)
            return 0
        lax.fori_loop(0, EB_W, _eb, 0)
        plsc.subcore_barrier()
        _node_epilogue()
        plsc.subcore_barrier()
        return 0
    lax.fori_loop(0, 8, _key_pass, 0)

    for r in range(5):
        def _snap(c8, _):
            Tv[r, pl.ds(c8 * 16, 16)] = Pv[r, pl.ds(c8 * 16, 16)]
            Pv[r, pl.ds(c8 * 16, 16)] = jnp.zeros((16,), jnp.int32)
            return 0
        lax.fori_loop(0, 8, _snap, 0)
    pltpu.sync_copy(Tv, Ts.at[pl.ds(w * 8, 8), :])
    pltpu.sync_copy(Pv, Ps.at[pl.ds(w * 8, 8), :])
    plsc.subcore_barrier()
    pltpu.sync_copy(Ts, sigT)

    def _tie_pass(p, _):
        sv = 16 - 4 * p
        pltpu.sync_copy(Ps, pLocal)

        def _eb(eb, _):
            _edge_scan(eb, False, sv)
            return 0
        lax.fori_loop(0, EB_W, _eb, 0)
        plsc.subcore_barrier()
        _node_epilogue()
        plsc.subcore_barrier()
        return 0
    lax.fori_loop(0, 5, _tie_pass, 0)

    pltpu.sync_copy(Tv, t_hbm.at[pl.ds(w * 8, 8), :])
    pltpu.sync_copy(Pv, i_hbm.at[pl.ds(w * 8, 8), :])
    pltpu.sync_copy(degs.at[pl.ds(nbase, NPW)], rv)
    for r in range(5):
        def _pubdeg(c8, _):
            Pv[r, pl.ds(c8 * 16, 16)] = rv[pl.ds(r * 128 + c8 * 16, 16)]
            return 0
        lax.fori_loop(0, 8, _pubdeg, 0)
    pltpu.sync_copy(Pv, deg_hbm.at[pl.ds(w * 8, 8), :])


def _flush_body(src_hbm, dst_hbm, key_hbm, t_hbm, i_hbm, deg_hbm, x_hbm,
                h_hbm, nm_hbm,
                sigT, dstv, keyv, srcv, pLocal, sidx, didx, degv8, hbuf,
                selsrc, seldst, rows, mrow, recipv,
                accs):
    w = lax.axis_index("s")
    nbase = w * NPW

    pltpu.sync_copy(h_hbm, hbuf)
    h = jnp.max(hbuf[0, pl.ds(0, 16)])

    for q in range(8):
        didx[0, pl.ds(q * 16, 16)] = q * 16 + _iota16()
    pltpu.sync_copy(t_hbm.at[didx.at[0]], sigT)
    pltpu.sync_copy(i_hbm.at[didx.at[0]], pLocal)

    sidx[0, pl.ds(0, 16)] = w * 8 + _iota16() % 8
    pltpu.sync_copy(deg_hbm.at[sidx.at[0, pl.ds(0, 8)]], degv8)

    def _load_edges(c, _):
        base = w * EB_W + c * 32
        sidx[0, pl.ds(0, 16)] = base + _iota16()
        sidx[0, pl.ds(16, 16)] = base + 16 + _iota16()
        pltpu.sync_copy(dst_hbm.at[sidx.at[0, pl.ds(0, 32)]],
                        dstv.at[pl.ds(c * 32, 32), :])
        pltpu.sync_copy(key_hbm.at[sidx.at[0, pl.ds(0, 32)]],
                        keyv.at[pl.ds(c * 32, 32), :])
        return 0
    lax.fori_loop(0, 5, _load_edges, 0)

    for j in range(16):
        for q in range(8):
            mrow[j, pl.ds(q * 16, 16)] = jnp.zeros((16,), jnp.float32)

    def _zero_acc(g, _):
        pltpu.sync_copy(mrow, accs.at[pl.ds(w * 160 + g * 16, 16), :])
        return 0
    lax.fori_loop(0, 10, _zero_acc, 0)

    @pl.when(w == NW - 1)
    def _zero_dump():
        pltpu.sync_copy(mrow.at[pl.ds(0, 8), :], accs.at[pl.ds(NQ, 8), :])
    plsc.subcore_barrier()

    def _chunk_flush(c, _):
        base = w * EB_W + c * 32
        sidx[0, pl.ds(0, 16)] = base + _iota16()
        sidx[0, pl.ds(16, 16)] = base + 16 + _iota16()
        pltpu.sync_copy(src_hbm.at[sidx.at[0, pl.ds(0, 32)]], srcv)

        def _prefill(jr, _):
            def _q(q, _):
                base16 = jr * 128 + q * 16 + _iota16()
                _rows(selsrc, jr, q, (base16 * 97 + w * 53) % N)
                _rows(seldst, jr, q, NQ + ((base16 + w * 16) % 8))
                return 0
            return lax.fori_loop(0, 8, _q, 0)
        lax.fori_loop(0, 32, _prefill, 0)

        def _compact(eb, o):
            row = c * 32 + eb

            def _q(q, o):
                k16 = _rowg(keyv, row, q)
                d16 = _rowg(dstv, row, q)
                s16 = _rowg(srcv, eb, q)
                t16 = _gat_pt(sigT, d16)
                i16v = _gat_pt(pLocal, d16)
                idx16 = (w * EB_W + row) * 128 + q * 16 + _iota16()
                keep = (k16 < t16) | ((k16 == t16) & (idx16 <= i16v))
                keep = keep & (d16 >= h * NQ) & (d16 < (h + 1) * NQ)
                ki = keep.astype(jnp.int32)
                pos = o + plsc.cumsum(ki) - 1
                rsel = lax.shift_right_logical(pos, _i16(7))
                csel = pos & 127
                plsc.store_scatter(selsrc, [rsel, csel], s16, mask=keep)
                plsc.store_scatter(seldst, [rsel, csel],
                                   d16 - h * NQ, mask=keep)
                return o + jnp.sum(ki)
            return lax.fori_loop(0, 8, _q, o)
        o = lax.fori_loop(0, 32, _compact, jnp.int32(0))

        nb = lax.shift_right_logical(o + 127, 7)

        def _flush(b, _):
            def _q(q, _):
                sidx[0, pl.ds(q * 16, 16)] = _rowg(selsrc, b, q)
                didx[0, pl.ds(q * 16, 16)] = _rowg(seldst, b, q)
                return 0
            lax.fori_loop(0, 8, _q, 0)
            pltpu.sync_copy(x_hbm.at[sidx.at[0]], rows)
            pltpu.sync_copy(rows, accs.at[didx.at[0]], add=True)
            return 0
        lax.fori_loop(0, nb, _flush, 0)
        return 0
    lax.fori_loop(0, 5, _chunk_flush, 0)
    plsc.subcore_barrier()

    @pl.when((w >= h * 4) & (w < (h + 1) * 4))
    def _mean_half():
        z16 = jnp.zeros((16,), jnp.int32)
        for r in range(5):
            loff = nbase - h * NQ + r * 128

            def _mean(c8, _):
                pltpu.sync_copy(accs.at[pl.ds(loff + c8 * 16, 16), :], mrow)
                d16 = degv8[r, pl.ds(c8 * 16, 16)]
                cnt = jnp.maximum(lax.shift_right_logical(d16, _i16(1)), 1)
                recipv[0, pl.ds(0, 16)] = 1.0 / cnt.astype(jnp.float32)

                def _j(j, _):
                    rj = plsc.load_gather(recipv, [z16, _i16(j)])

                    def _q2(q, _):
                        v = _rowg(mrow, j, q) * rj
                        plsc.store_scatter(
                            rows, [_i16(c8 * 16 + j), q * 16 + _iota16()], v)
                        return 0
                    return lax.fori_loop(0, 8, _q2, 0)
                lax.fori_loop(0, 16, _j, 0)
                return 0
            lax.fori_loop(0, 8, _mean, 0)
            for q in range(8):
                didx[0, pl.ds(q * 16, 16)] = loff + q * 16 + _iota16()
            pltpu.sync_copy(rows, nm_hbm.at[didx.at[0]])
    plsc.subcore_barrier()


_MESH = dict(core_axis_name="c", subcore_axis_name="s", num_cores=1)


@jax.jit
def _sc_select(sig2d, src2d, dst2d):
    f = pl.kernel(
        _select_body,
        compiler_params=pltpu.CompilerParams(needs_layout_passes=False),
        out_type=[
            jax.ShapeDtypeStruct((ER, 128), jnp.int32),
            jax.ShapeDtypeStruct((128, 128), jnp.int32),
            jax.ShapeDtypeStruct((128, 128), jnp.int32),
            jax.ShapeDtypeStruct((128, 128), jnp.int32),
        ],
        mesh=plsc.VectorSubcoreMesh(**_MESH),
        scratch_types=[
            pltpu.VMEM((128, 128), jnp.int32),
            pltpu.VMEM((EB_W, 128), jnp.int32),
            pltpu.VMEM((EB_W, 128), jnp.int32),
            pltpu.VMEM((32, 128), jnp.int32),
            pltpu.VMEM((128, 128), jnp.int32),
            pltpu.VMEM((1, 128), jnp.int32),
            pltpu.VMEM((128,), jnp.int32),
            pltpu.VMEM((16 * NPW,), jnp.int32),
            pltpu.VMEM((NPW,), jnp.int32),
            pltpu.VMEM((NPW,), jnp.int32),
            pltpu.VMEM((8, 128), jnp.int32),
            pltpu.VMEM((8, 128), jnp.int32),
            pltpu.VMEM_SHARED((HSIZE,), jnp.int32),
            pltpu.VMEM_SHARED((NPAD,), jnp.int32),
            pltpu.VMEM_SHARED((128, 128), jnp.int32),
            pltpu.VMEM_SHARED((128, 128), jnp.int32),
            pltpu.VMEM_SHARED((NR, 128), jnp.int32),
        ],
    )
    return f(sig2d, src2d, dst2d)


@jax.jit
def _sc_flush(src2d, dst2d, key2d, t2d, i2d, deg2d, x_big, h_arr):
    f = pl.kernel(
        _flush_body,
        compiler_params=pltpu.CompilerParams(needs_layout_passes=False),
        out_type=jax.ShapeDtypeStruct((NQ, D), jnp.float32),
        mesh=plsc.VectorSubcoreMesh(**_MESH),
        scratch_types=[
            pltpu.VMEM((128, 128), jnp.int32),
            pltpu.VMEM((EB_W, 128), jnp.int32),
            pltpu.VMEM((EB_W, 128), jnp.int32),
            pltpu.VMEM((32, 128), jnp.int32),
            pltpu.VMEM((128, 128), jnp.int32),
            pltpu.VMEM((1, 128), jnp.int32),
            pltpu.VMEM((1, 128), jnp.int32),
            pltpu.VMEM((8, 128), jnp.int32),
            pltpu.VMEM((8, 128), jnp.int32),
            pltpu.VMEM((32, 128), jnp.int32),
            pltpu.VMEM((32, 128), jnp.int32),
            pltpu.VMEM((128, 128), jnp.float32),
            pltpu.VMEM((16, 128), jnp.float32),
            pltpu.VMEM((1, 128), jnp.float32),
            pltpu.VMEM_SHARED((NQ + 8, D), jnp.float32),
        ],
    )
    return f(src2d, dst2d, key2d, t2d, i2d, deg2d, x_big, h_arr)


def _dist_body(x_ref, wd_ref, bd_ref, s_ref, sb_ref):
    s = jnp.dot(x_ref[...], wd_ref[...], preferred_element_type=jnp.float32)
    s = s + bd_ref[...]
    s_ref[...] = s
    s0 = s[:, 0:1]
    sig = 1.0 / (1.0 + jnp.exp(-s0))
    sb_ref[...] = lax.bitcast_convert_type(
        jnp.broadcast_to(sig, (s.shape[0], 128)), jnp.int32)


def _dist_matmul(x, W_dist_pad, b_dist_pad):
    n, d = x.shape
    blk = 1000
    return pl.pallas_call(
        _dist_body,
        grid=(n // blk,),
        in_specs=[
            pl.BlockSpec((blk, d), lambda i: (i, 0)),
            pl.BlockSpec(W_dist_pad.shape, lambda i: (0, 0)),
            pl.BlockSpec(b_dist_pad.shape, lambda i: (0, 0)),
        ],
        out_specs=[
            pl.BlockSpec((blk, 128), lambda i: (i, 0)),
            pl.BlockSpec((blk, 128), lambda i: (i, 0)),
        ],
        out_shape=[
            jax.ShapeDtypeStruct((n, 128), jnp.float32),
            jax.ShapeDtypeStruct((n, 128), jnp.int32),
        ],
    )(x, W_dist_pad, b_dist_pad)


def _final_body(x_ref, nm_ref, ws_ref, wn_ref, bs_ref, wo_ref, bo_ref, out_ref):
    h = jnp.dot(x_ref[...], ws_ref[...], preferred_element_type=jnp.float32)
    h += jnp.dot(nm_ref[...], wn_ref[...], preferred_element_type=jnp.float32)
    h += bs_ref[...]
    h = jnp.maximum(h, 0.0)
    out_ref[...] = jnp.dot(h, wo_ref[...],
                           preferred_element_type=jnp.float32) + bo_ref[...]


def _final_matmuls(x, neigh_mean, W_self, W_neigh, b_sage2, W_out_pad, b_out_pad):
    n, d = x.shape
    blk = 1000
    return pl.pallas_call(
        _final_body,
        grid=(n // blk,),
        in_specs=[
            pl.BlockSpec((blk, d), lambda i: (i, 0)),
            pl.BlockSpec((blk, d), lambda i: (i, 0)),
            pl.BlockSpec(W_self.shape, lambda i: (0, 0)),
            pl.BlockSpec(W_neigh.shape, lambda i: (0, 0)),
            pl.BlockSpec(b_sage2.shape, lambda i: (0, 0)),
            pl.BlockSpec(W_out_pad.shape, lambda i: (0, 0)),
            pl.BlockSpec(b_out_pad.shape, lambda i: (0, 0)),
        ],
        out_specs=pl.BlockSpec((blk, 128), lambda i: (i, 0)),
        out_shape=jax.ShapeDtypeStruct((n, 128), jnp.float32),
    )(x, neigh_mean, W_self, W_neigh, b_sage2, W_out_pad, b_out_pad)


def kernel(x, edge_index, W_dist, b_dist, W_self, W_neigh, b_sage, W_out, b_out):
    e = edge_index.shape[1]

    Wd_pad = jnp.zeros((D, 128), jnp.float32).at[:, :2].set(W_dist)
    bd_pad = jnp.zeros((1, 128), jnp.float32).at[0, :2].set(b_dist)
    s_pad, sig_bits_col = _dist_matmul(x, Wd_pad, bd_pad)
    sig2d = jnp.pad(sig_bits_col[:, 0], (0, NPAD - N)).reshape(NR, 128)

    src = edge_index[0]
    dst = edge_index[1]
    pe = EPAD - e
    ar = jnp.arange(pe, dtype=jnp.int32)
    src2d = jnp.tile(jnp.concatenate([src, ar % N]).reshape(ER, 128), (7, 1))
    dst2d = jnp.tile(
        jnp.concatenate([dst, N + (ar % (NPAD - N))]).reshape(ER, 128), (7, 1))

    key2d, t2d, i2d, deg2d = _sc_select(sig2d, src2d, dst2d)

    x_big = jnp.concatenate([x, x])
    parts = []
    dep = jnp.zeros((1, 1), jnp.int32)
    for hq in range(4):
        h_arr = jnp.full((8, 128), hq, jnp.int32) + dep[0, 0]
        part = _sc_flush(src2d, dst2d, key2d, t2d, i2d, deg2d, x_big, h_arr)
        parts.append(part)
        dep = (part[:1, :1] * 0.0).astype(jnp.int32)
    nm = jnp.concatenate(parts)[:N]

    b_sage2 = jnp.broadcast_to(b_sage[None, :], (1, 128))
    W_out_pad = jnp.zeros((128, 128), jnp.float32).at[:, :2].set(W_out)
    b_out_pad = jnp.zeros((1, 128), jnp.float32).at[0, :2].set(b_out)
    out = _final_matmuls(x, nm, W_self, W_neigh, b_sage2, W_out_pad, b_out_pad)
    return (out[:, :2], s_pad[:, :2])
